# Initial kernel scaffold; baseline (speedup 1.0000x reference)
import jax
import jax.numpy as jnp
from jax.experimental import pallas as pl


def kernel(node_fea, clu_label, center_fea, mask_nodes, mask_weight, sort_idx_rst):
    raise NotImplementedError("write your pallas kernel here")


# speedup vs baseline: 11.8037x; 11.8037x over previous
import jax
import jax.numpy as jnp
from jax import lax
from jax.experimental import pallas as pl
from jax.experimental.pallas import tpu as pltpu
from jax.experimental.pallas import tpu_sc as plsc

N, D, K, M, S = 10000, 256, 8, 1000, 1250
NEDGE = 125
THPOS = 12
EPS_PD = 1e-6
EPS_COS = 1e-8

NW = 32
BPAD = 1024
GPT = BPAD // NW
NPAD = 10240
RPT = NPAD // NW
MPAD = 1024

NB = 10
BLK = N // NB


def _sc_body(node_hbm, idx_hbm, mask_hbm, f_hbm, flags_hbm,
             idx_v, rows_v, mask_v, flag_v, sem):
    c = lax.axis_index("c")
    s = lax.axis_index("s")
    wid = s * 2 + c

    gbase = wid * GPT
    pltpu.sync_copy(idx_hbm.at[pl.ds(gbase, GPT)], idx_v)
    pltpu.async_copy(node_hbm.at[idx_v], rows_v, sem).wait()
    pltpu.sync_copy(rows_v, f_hbm.at[pl.ds(gbase, GPT)])

    fbase = wid * RPT
    zeros16 = jnp.zeros((16,), jnp.float32)
    for i in range(RPT // 16):
        flag_v[pl.ds(i * 16, 16)] = zeros16
    pltpu.sync_copy(mask_hbm, mask_v)
    ones16 = jnp.ones((16,), jnp.float32)
    for i in range(MPAD // 16):
        mv = mask_v[pl.ds(i * 16, 16)]
        off = mv - fbase
        valid = (off >= 0) & (off < RPT)
        offc = jnp.minimum(jnp.maximum(off, 0), RPT - 1)
        plsc.store_scatter(flag_v, [offc], ones16, mask=valid)
    pltpu.sync_copy(flag_v, flags_hbm.at[pl.ds(fbase, RPT)])


def _tc_body(nf_ref, lab_ref, wts_ref, cen_ref, f_ref, out_ref, acc_ref):
    b = pl.program_id(0)

    @pl.when(b == 0)
    def _init():
        acc_ref[0, 0] = jnp.float32(0.0)

    x = nf_ref[...]
    lab = lab_ref[0, 0, :]
    wts = wts_ref[0, 0, :]
    cen = cen_ref[...]
    oh = (lab[:, None] == lax.broadcasted_iota(jnp.int32, (BLK, K), 1)
          ).astype(jnp.float32)
    csel = lax.dot_general(oh, cen, (((1,), (0,)), ((), ())),
                           preferred_element_type=jnp.float32)
    diff = x - csel + EPS_PD
    d = jnp.sqrt(jnp.sum(diff * diff, axis=1))
    acc_ref[0, 0] += jnp.sum(d * wts)

    @pl.when(b == NB - 1)
    def _fin():
        F = f_ref[0:K * NEDGE, :].reshape(K, NEDGE, D)
        fn = jnp.maximum(jnp.sqrt(jnp.sum(F * F, axis=2)), EPS_COS)
        cd = cen[:, None, :] - cen[None, :, :]
        tn = jnp.maximum(jnp.sqrt(jnp.sum(cd * cd, axis=2)), EPS_COS)

        means = []
        cnts = []
        for i in range(K):
            Fi = F[i]
            g = lax.dot_general(cen, Fi, (((1,), (1,)), ((), ())),
                                preferred_element_type=jnp.float32)
            num = g[i, :][None, :] - g
            cos = num / (tn[i, :][:, None] * fn[i, :][None, :])
            less = (cos[:, None, :] < cos[:, :, None]).astype(jnp.float32)
            cnt_less = jnp.sum(less, axis=2)
            keep = (cnt_less > jnp.float32(THPOS)).astype(jnp.float32)
            cnt = jnp.sum(keep, axis=1)
            ssum = lax.dot_general(keep, Fi, (((1,), (0,)), ((), ())),
                                   preferred_element_type=jnp.float32)
            mean = ssum / jnp.maximum(cnt, 1.0)[:, None]
            means.append(mean)
            cnts.append(cnt)

        L2 = jnp.float32(0.0)
        for i in range(K):
            for j in range(i + 1, K):
                dd = means[i][j] - means[j][i] + EPS_PD
                dist = jnp.sqrt(jnp.sum(dd * dd))
                ok = (cnts[i][j] > 0.0) & (cnts[j][i] > 0.0)
                L2 = L2 + jnp.where(ok, dist, jnp.float32(0.0))

        total = acc_ref[0, 0] - L2
        out_ref[...] = jnp.broadcast_to(total, (1, 1))


def kernel(node_fea, clu_label, center_fea, mask_nodes, mask_weight, sort_idx_rst):
    node_fea = node_fea.astype(jnp.float32)
    center_fea = center_fea.astype(jnp.float32)

    idx = sort_idx_rst[:, S - NEDGE:].astype(jnp.int32).reshape(-1)
    idx = jnp.concatenate([idx, jnp.zeros((BPAD - K * NEDGE,), jnp.int32)])
    mask = jnp.concatenate([mask_nodes.astype(jnp.int32),
                            jnp.full((MPAD - M,), -1, jnp.int32)])

    sc_call = pl.kernel(
        _sc_body,
        out_type=[jax.ShapeDtypeStruct((BPAD, D), jnp.float32),
                  jax.ShapeDtypeStruct((NPAD,), jnp.float32)],
        mesh=plsc.VectorSubcoreMesh(core_axis_name="c", subcore_axis_name="s"),
        compiler_params=pltpu.CompilerParams(needs_layout_passes=False),
        scratch_types=[
            pltpu.VMEM((GPT,), jnp.int32),
            pltpu.VMEM((GPT, D), jnp.float32),
            pltpu.VMEM((MPAD,), jnp.int32),
            pltpu.VMEM((RPT,), jnp.float32),
            pltpu.SemaphoreType.DMA,
        ],
    )
    F, flags = sc_call(node_fea, idx, mask)

    mwf = jnp.asarray(mask_weight, jnp.float32)
    wts = 1.0 + (1.0 + mwf) * flags[:N]

    labs = clu_label.astype(jnp.int32).reshape(NB, 1, BLK)
    wts_r = wts.reshape(NB, 1, BLK)

    out = pl.pallas_call(
        _tc_body,
        grid=(NB,),
        in_specs=[
            pl.BlockSpec((BLK, D), lambda b: (b, 0)),
            pl.BlockSpec((1, 1, BLK), lambda b: (b, 0, 0)),
            pl.BlockSpec((1, 1, BLK), lambda b: (b, 0, 0)),
            pl.BlockSpec((K, D), lambda b: (0, 0)),
            pl.BlockSpec((BPAD, D), lambda b: (0, 0)),
        ],
        out_specs=pl.BlockSpec((1, 1), lambda b: (0, 0)),
        out_shape=jax.ShapeDtypeStruct((1, 1), jnp.float32),
        scratch_shapes=[pltpu.SMEM((1, 1), jnp.float32)],
    )(node_fea, labs, wts_r, center_fea, F)
    return out.reshape(1)


# speedup vs baseline: 12.0488x; 1.0208x over previous
import jax
import jax.numpy as jnp
from jax import lax
from jax.experimental import pallas as pl
from jax.experimental.pallas import tpu as pltpu
from jax.experimental.pallas import tpu_sc as plsc

N, D, K, M, S = 10000, 256, 8, 1000, 1250
NEDGE = 125
THPOS = 12
EPS_PD = 1e-6
EPS_COS = 1e-8

NW = 32
BPAD = 1024
GPT = BPAD // NW
FW = 25
RPT = N // FW
MPAD = 1024

NB = 10
BLK = N // NB


def _sc_body(node_hbm, idx_hbm, mask_hbm, wval_hbm, f_hbm, wts_hbm,
             idx_v, rows_v, mask_v, flag_v, wval_v, sem):
    c = lax.axis_index("c")
    s = lax.axis_index("s")
    wid = s * 2 + c

    gbase = wid * GPT
    pltpu.sync_copy(idx_hbm.at[pl.ds(gbase, GPT)], idx_v)
    pltpu.async_copy(node_hbm.at[idx_v], rows_v, sem).wait()
    pltpu.sync_copy(rows_v, f_hbm.at[pl.ds(gbase, GPT)])

    @pl.when(wid < FW)
    def _flags():
        fbase = wid * RPT
        pltpu.sync_copy(mask_hbm, mask_v)
        pltpu.sync_copy(wval_hbm, wval_v)
        ones16 = jnp.ones((16,), jnp.float32)
        for i in range(RPT // 16):
            flag_v[pl.ds(i * 16, 16)] = ones16
        wv = wval_v[...]
        for i in range(MPAD // 16):
            mv = mask_v[pl.ds(i * 16, 16)]
            off = mv - fbase
            valid = (off >= 0) & (off < RPT)
            offc = jnp.minimum(jnp.maximum(off, 0), RPT - 1)
            plsc.store_scatter(flag_v, [offc], wv, mask=valid)
        pltpu.sync_copy(flag_v, wts_hbm.at[pl.ds(fbase, RPT)])


def _tc_sweep_body(nf_ref, lab_ref, cen_ref, d_ref):
    x = nf_ref[...]
    lab = lab_ref[0, 0, :]
    cen = cen_ref[...]
    oh = (lab[:, None] == lax.broadcasted_iota(jnp.int32, (BLK, K), 1)
          ).astype(jnp.float32)
    csel = lax.dot_general(oh, cen, (((1,), (0,)), ((), ())),
                           preferred_element_type=jnp.float32)
    diff = x - csel + EPS_PD
    d = jnp.sqrt(jnp.sum(diff * diff, axis=1))
    d_ref[...] = d.reshape(1, 1, BLK)


def _tc_combine_body(d_ref, wts_ref, cen_ref, f_ref, out_ref):
    inner = jnp.sum(d_ref[...] * wts_ref[...])

    cen = cen_ref[...]
    F = f_ref[0:K * NEDGE, :].reshape(K, NEDGE, D)
    fn = jnp.maximum(jnp.sqrt(jnp.sum(F * F, axis=2)), EPS_COS)
    cd = cen[:, None, :] - cen[None, :, :]
    tn = jnp.maximum(jnp.sqrt(jnp.sum(cd * cd, axis=2)), EPS_COS)

    means = []
    cnts = []
    for i in range(K):
        Fi = F[i]
        g = lax.dot_general(cen, Fi, (((1,), (1,)), ((), ())),
                            preferred_element_type=jnp.float32)
        num = g[i, :][None, :] - g
        cos = num / (tn[i, :][:, None] * fn[i, :][None, :])
        less = (cos[:, None, :] < cos[:, :, None]).astype(jnp.float32)
        cnt_less = jnp.sum(less, axis=2)
        keep = (cnt_less > jnp.float32(THPOS)).astype(jnp.float32)
        cnt = jnp.sum(keep, axis=1)
        ssum = lax.dot_general(keep, Fi, (((1,), (0,)), ((), ())),
                               preferred_element_type=jnp.float32)
        mean = ssum / jnp.maximum(cnt, 1.0)[:, None]
        means.append(mean)
        cnts.append(cnt)

    L2 = jnp.float32(0.0)
    for i in range(K):
        for j in range(i + 1, K):
            dd = means[i][j] - means[j][i] + EPS_PD
            dist = jnp.sqrt(jnp.sum(dd * dd))
            ok = (cnts[i][j] > 0.0) & (cnts[j][i] > 0.0)
            L2 = L2 + jnp.where(ok, dist, jnp.float32(0.0))

    out_ref[...] = jnp.broadcast_to(inner - L2, (1, 1))


def kernel(node_fea, clu_label, center_fea, mask_nodes, mask_weight, sort_idx_rst):
    node_fea = node_fea.astype(jnp.float32)
    center_fea = center_fea.astype(jnp.float32)

    idx = sort_idx_rst[:, S - NEDGE:].astype(jnp.int32).reshape(-1)
    idx = jnp.concatenate([idx, jnp.zeros((BPAD - K * NEDGE,), jnp.int32)])
    mask = jnp.concatenate([mask_nodes.astype(jnp.int32),
                            jnp.full((MPAD - M,), -1, jnp.int32)])
    mwf = jnp.asarray(mask_weight, jnp.float32)
    wval = jnp.full((16,), 1.0, jnp.float32) + mwf + 1.0

    sc_call = pl.kernel(
        _sc_body,
        out_type=[jax.ShapeDtypeStruct((BPAD, D), jnp.float32),
                  jax.ShapeDtypeStruct((N,), jnp.float32)],
        mesh=plsc.VectorSubcoreMesh(core_axis_name="c", subcore_axis_name="s"),
        compiler_params=pltpu.CompilerParams(needs_layout_passes=False),
        scratch_types=[
            pltpu.VMEM((GPT,), jnp.int32),
            pltpu.VMEM((GPT, D), jnp.float32),
            pltpu.VMEM((MPAD,), jnp.int32),
            pltpu.VMEM((RPT,), jnp.float32),
            pltpu.VMEM((16,), jnp.float32),
            pltpu.SemaphoreType.DMA,
        ],
    )
    F, wts = sc_call(node_fea, idx, mask, wval)

    labs = clu_label.astype(jnp.int32).reshape(NB, 1, BLK)

    d = pl.pallas_call(
        _tc_sweep_body,
        grid=(NB,),
        in_specs=[
            pl.BlockSpec((BLK, D), lambda b: (b, 0)),
            pl.BlockSpec((1, 1, BLK), lambda b: (b, 0, 0)),
            pl.BlockSpec((K, D), lambda b: (0, 0)),
        ],
        out_specs=pl.BlockSpec((1, 1, BLK), lambda b: (b, 0, 0)),
        out_shape=jax.ShapeDtypeStruct((NB, 1, BLK), jnp.float32),
    )(node_fea, labs, center_fea)

    wts_r = wts.reshape(NB, 1, BLK)

    out = pl.pallas_call(
        _tc_combine_body,
        in_specs=[
            pl.BlockSpec((NB, 1, BLK), lambda: (0, 0, 0)),
            pl.BlockSpec((NB, 1, BLK), lambda: (0, 0, 0)),
            pl.BlockSpec((K, D), lambda: (0, 0)),
            pl.BlockSpec((BPAD, D), lambda: (0, 0)),
        ],
        out_specs=pl.BlockSpec((1, 1), lambda: (0, 0)),
        out_shape=jax.ShapeDtypeStruct((1, 1), jnp.float32),
    )(d, wts_r, center_fea, F)
    return out.reshape(1)


# speedup vs baseline: 12.3676x; 1.0265x over previous
import jax
import jax.numpy as jnp
from jax import lax
from jax.experimental import pallas as pl
from jax.experimental.pallas import tpu as pltpu
from jax.experimental.pallas import tpu_sc as plsc

N, D, K, M, S = 10000, 256, 8, 1000, 1250
NEDGE = 125
THPOS = 12
EPS_PD = 1e-6
EPS_COS = 1e-8

NW = 32
BPAD = 1024
GPT = BPAD // NW
FW = 5
RPT = 2000
MPAD = 1024

NB = 5
BLK = N // NB


def _sc_body(node_hbm, idx_hbm, mask_hbm, wval_hbm, f_hbm, wts_hbm,
             idx_v, rows_v, mask_v, flag_v, wval_v, sem):
    c = lax.axis_index("c")
    s = lax.axis_index("s")
    wid = s * 2 + c

    gbase = wid * GPT
    pltpu.sync_copy(idx_hbm.at[pl.ds(gbase, GPT)], idx_v)
    pltpu.async_copy(node_hbm.at[idx_v], rows_v, sem).wait()
    pltpu.sync_copy(rows_v, f_hbm.at[pl.ds(gbase, GPT)])

    @pl.when(wid < FW)
    def _flags():
        fbase = wid * RPT
        pltpu.sync_copy(mask_hbm, mask_v)
        pltpu.sync_copy(wval_hbm, wval_v)
        ones16 = jnp.ones((16,), jnp.float32)
        for i in range(RPT // 16):
            flag_v[pl.ds(i * 16, 16)] = ones16
        wv = wval_v[...]
        for i in range(MPAD // 16):
            mv = mask_v[pl.ds(i * 16, 16)]
            off = mv - fbase
            valid = (off >= 0) & (off < RPT)
            offc = jnp.minimum(jnp.maximum(off, 0), RPT - 1)
            plsc.store_scatter(flag_v, [offc], wv, mask=valid)
        pltpu.sync_copy(flag_v, wts_hbm.at[wid, 0, pl.ds(0, RPT)])


def _tc_body(nf_ref, lab_ref, cen_ref, wts_ref, f_any, out_ref,
             dscr, f_vmem, fsem):
    b = pl.program_id(0)

    @pl.when(b == 0)
    def _start_f():
        pltpu.make_async_copy(f_any, f_vmem, fsem).start()

    x = nf_ref[...]
    lab = lab_ref[0, 0, :]
    cen = cen_ref[...]
    oh = (lab[:, None] == lax.broadcasted_iota(jnp.int32, (BLK, K), 1)
          ).astype(jnp.float32)
    csel = lax.dot_general(oh, cen, (((1,), (0,)), ((), ())),
                           preferred_element_type=jnp.float32)
    diff = x - csel + EPS_PD
    d = jnp.sqrt(jnp.sum(diff * diff, axis=1))
    dscr[pl.ds(b, 1), :, :] = d.reshape(1, 1, BLK)

    @pl.when(b == NB - 1)
    def _fin():
        inner = jnp.sum(dscr[...].reshape(NB, BLK) * wts_ref[...].reshape(NB, BLK))

        pltpu.make_async_copy(f_any, f_vmem, fsem).wait()
        F = f_vmem[0:K * NEDGE, :].reshape(K, NEDGE, D)
        fn = jnp.maximum(jnp.sqrt(jnp.sum(F * F, axis=2)), EPS_COS)
        cd = cen[:, None, :] - cen[None, :, :]
        tn = jnp.maximum(jnp.sqrt(jnp.sum(cd * cd, axis=2)), EPS_COS)

        means = []
        cnts = []
        for i in range(K):
            Fi = F[i]
            g = lax.dot_general(cen, Fi, (((1,), (1,)), ((), ())),
                                preferred_element_type=jnp.float32)
            num = g[i, :][None, :] - g
            cos = num / (tn[i, :][:, None] * fn[i, :][None, :])
            less = (cos[:, None, :] < cos[:, :, None]).astype(jnp.float32)
            cnt_less = jnp.sum(less, axis=2)
            keep = (cnt_less > jnp.float32(THPOS)).astype(jnp.float32)
            cnt = jnp.sum(keep, axis=1)
            ssum = lax.dot_general(keep, Fi, (((1,), (0,)), ((), ())),
                                   preferred_element_type=jnp.float32)
            mean = ssum / jnp.maximum(cnt, 1.0)[:, None]
            means.append(mean)
            cnts.append(cnt)

        L2 = jnp.float32(0.0)
        for i in range(K):
            for j in range(i + 1, K):
                dd = means[i][j] - means[j][i] + EPS_PD
                dist = jnp.sqrt(jnp.sum(dd * dd))
                ok = (cnts[i][j] > 0.0) & (cnts[j][i] > 0.0)
                L2 = L2 + jnp.where(ok, dist, jnp.float32(0.0))

        out_ref[...] = jnp.broadcast_to(inner - L2, (1, 1))


def kernel(node_fea, clu_label, center_fea, mask_nodes, mask_weight, sort_idx_rst):
    node_fea = node_fea.astype(jnp.float32)
    center_fea = center_fea.astype(jnp.float32)

    idx = sort_idx_rst[:, S - NEDGE:].astype(jnp.int32).reshape(-1)
    idx = jnp.concatenate([idx, jnp.zeros((BPAD - K * NEDGE,), jnp.int32)])
    mask = jnp.concatenate([mask_nodes.astype(jnp.int32),
                            jnp.full((MPAD - M,), -1, jnp.int32)])
    mwf = jnp.asarray(mask_weight, jnp.float32)
    wval = jnp.full((16,), 2.0, jnp.float32) + mwf

    sc_call = pl.kernel(
        _sc_body,
        out_type=[jax.ShapeDtypeStruct((BPAD, D), jnp.float32),
                  jax.ShapeDtypeStruct((NB, 1, BLK), jnp.float32)],
        mesh=plsc.VectorSubcoreMesh(core_axis_name="c", subcore_axis_name="s"),
        compiler_params=pltpu.CompilerParams(needs_layout_passes=False),
        scratch_types=[
            pltpu.VMEM((GPT,), jnp.int32),
            pltpu.VMEM((GPT, D), jnp.float32),
            pltpu.VMEM((MPAD,), jnp.int32),
            pltpu.VMEM((RPT,), jnp.float32),
            pltpu.VMEM((16,), jnp.float32),
            pltpu.SemaphoreType.DMA,
        ],
    )
    F, wts = sc_call(node_fea, idx, mask, wval)

    labs = clu_label.astype(jnp.int32).reshape(NB, 1, BLK)

    out = pl.pallas_call(
        _tc_body,
        grid=(NB,),
        in_specs=[
            pl.BlockSpec((BLK, D), lambda b: (b, 0)),
            pl.BlockSpec((1, 1, BLK), lambda b: (b, 0, 0)),
            pl.BlockSpec((K, D), lambda b: (0, 0)),
            pl.BlockSpec((NB, 1, BLK), lambda b: (0, 0, 0)),
            pl.BlockSpec(memory_space=pl.ANY),
        ],
        out_specs=pl.BlockSpec((1, 1), lambda b: (0, 0)),
        out_shape=jax.ShapeDtypeStruct((1, 1), jnp.float32),
        scratch_shapes=[
            pltpu.VMEM((NB, 1, BLK), jnp.float32),
            pltpu.VMEM((BPAD, D), jnp.float32),
            pltpu.SemaphoreType.DMA,
        ],
    )(node_fea, labs, center_fea, wts, F)
    return out.reshape(1)


# speedup vs baseline: 12.8977x; 1.0429x over previous
import jax
import jax.numpy as jnp
from jax import lax
from jax.experimental import pallas as pl
from jax.experimental.pallas import tpu as pltpu
from jax.experimental.pallas import tpu_sc as plsc

N, D, K, M, S = 10000, 256, 8, 1000, 1250
NEDGE = 125
THPOS = 12
EPS_PD = 1e-6
EPS_COS = 1e-8

NW = 32
BPAD = 1024
GPT = BPAD // NW
FW = 5
RPT = 2000
MPAD = 1024

NB = 5
BLK = N // NB


def _sc_body(node_hbm, idx_hbm, mask_hbm, wval_hbm, f_hbm, wts_hbm,
             idx_v, rows_v, mask_v, flag_v, wval_v,
             sem_i, sem_m, sem_w, sem_g):
    c = lax.axis_index("c")
    s = lax.axis_index("s")
    wid = s * 2 + c

    gbase = wid * GPT
    pltpu.async_copy(idx_hbm.at[pl.ds(gbase, GPT)], idx_v, sem_i).wait()
    gath = pltpu.async_copy(node_hbm.at[idx_v], rows_v, sem_g)

    @pl.when(wid < FW)
    def _flags():
        fbase = wid * RPT
        cm = pltpu.async_copy(mask_hbm, mask_v, sem_m)
        cw = pltpu.async_copy(wval_hbm, wval_v, sem_w)
        ones16 = jnp.ones((16,), jnp.float32)
        for i in range(RPT // 16):
            flag_v[pl.ds(i * 16, 16)] = ones16
        cm.wait()
        cw.wait()
        wv = wval_v[...]
        for i in range(MPAD // 16):
            mv = mask_v[pl.ds(i * 16, 16)]
            off = mv - fbase
            valid = (off >= 0) & (off < RPT)
            offc = jnp.minimum(jnp.maximum(off, 0), RPT - 1)
            plsc.store_scatter(flag_v, [offc], wv, mask=valid)
        pltpu.sync_copy(flag_v, wts_hbm.at[wid, 0, pl.ds(0, RPT)])

    gath.wait()
    pltpu.sync_copy(rows_v, f_hbm.at[pl.ds(gbase, GPT)])


def _tc_body(nf_ref, lab_ref, cen_ref, wts_ref, f_any, out_ref,
             dscr, f_vmem, fsem):
    b = pl.program_id(0)

    @pl.when(b == 0)
    def _start_f():
        pltpu.make_async_copy(f_any, f_vmem, fsem).start()

    x = nf_ref[...]
    lab = lab_ref[0, 0, :]
    cen = cen_ref[...]
    oh = (lab[:, None] == lax.broadcasted_iota(jnp.int32, (BLK, K), 1)
          ).astype(jnp.float32)
    csel = lax.dot_general(oh, cen, (((1,), (0,)), ((), ())),
                           preferred_element_type=jnp.float32)
    diff = x - csel + EPS_PD
    d = jnp.sqrt(jnp.sum(diff * diff, axis=1))
    dscr[pl.ds(b, 1), :, :] = d.reshape(1, 1, BLK)

    @pl.when(b == NB - 1)
    def _fin():
        inner = jnp.sum(dscr[...].reshape(NB, BLK) * wts_ref[...].reshape(NB, BLK))

        pltpu.make_async_copy(f_any, f_vmem, fsem).wait()
        F = f_vmem[0:K * NEDGE, :].reshape(K, NEDGE, D)
        fn = jnp.maximum(jnp.sqrt(jnp.sum(F * F, axis=2)), EPS_COS)
        cd = cen[:, None, :] - cen[None, :, :]
        tn = jnp.maximum(jnp.sqrt(jnp.sum(cd * cd, axis=2)), EPS_COS)

        means = []
        cnts = []
        for i in range(K):
            Fi = F[i]
            g = lax.dot_general(cen, Fi, (((1,), (1,)), ((), ())),
                                preferred_element_type=jnp.float32)
            num = g[i, :][None, :] - g
            cos = num / (tn[i, :][:, None] * fn[i, :][None, :])
            less = (cos[:, None, :] < cos[:, :, None]).astype(jnp.float32)
            cnt_less = jnp.sum(less, axis=2)
            keep = (cnt_less > jnp.float32(THPOS)).astype(jnp.float32)
            cnt = jnp.sum(keep, axis=1)
            ssum = lax.dot_general(keep, Fi, (((1,), (0,)), ((), ())),
                                   preferred_element_type=jnp.float32)
            mean = ssum / jnp.maximum(cnt, 1.0)[:, None]
            means.append(mean)
            cnts.append(cnt)

        L2 = jnp.float32(0.0)
        for i in range(K):
            for j in range(i + 1, K):
                dd = means[i][j] - means[j][i] + EPS_PD
                dist = jnp.sqrt(jnp.sum(dd * dd))
                ok = (cnts[i][j] > 0.0) & (cnts[j][i] > 0.0)
                L2 = L2 + jnp.where(ok, dist, jnp.float32(0.0))

        out_ref[...] = jnp.broadcast_to(inner - L2, (1, 1))


def kernel(node_fea, clu_label, center_fea, mask_nodes, mask_weight, sort_idx_rst):
    node_fea = node_fea.astype(jnp.float32)
    center_fea = center_fea.astype(jnp.float32)

    idx = sort_idx_rst[:, S - NEDGE:].astype(jnp.int32).reshape(-1)
    idx = jnp.concatenate([idx, jnp.zeros((BPAD - K * NEDGE,), jnp.int32)])
    mask = jnp.concatenate([mask_nodes.astype(jnp.int32),
                            jnp.full((MPAD - M,), -1, jnp.int32)])
    mwf = jnp.asarray(mask_weight, jnp.float32)
    wval = jnp.full((16,), 2.0, jnp.float32) + mwf

    sc_call = pl.kernel(
        _sc_body,
        out_type=[jax.ShapeDtypeStruct((BPAD, D), jnp.float32),
                  jax.ShapeDtypeStruct((NB, 1, BLK), jnp.float32)],
        mesh=plsc.VectorSubcoreMesh(core_axis_name="c", subcore_axis_name="s"),
        compiler_params=pltpu.CompilerParams(needs_layout_passes=False),
        scratch_types=[
            pltpu.VMEM((GPT,), jnp.int32),
            pltpu.VMEM((GPT, D), jnp.float32),
            pltpu.VMEM((MPAD,), jnp.int32),
            pltpu.VMEM((RPT,), jnp.float32),
            pltpu.VMEM((16,), jnp.float32),
            pltpu.SemaphoreType.DMA,
            pltpu.SemaphoreType.DMA,
            pltpu.SemaphoreType.DMA,
            pltpu.SemaphoreType.DMA,
        ],
    )
    F, wts = sc_call(node_fea, idx, mask, wval)

    labs = clu_label.astype(jnp.int32).reshape(NB, 1, BLK)

    out = pl.pallas_call(
        _tc_body,
        grid=(NB,),
        in_specs=[
            pl.BlockSpec((BLK, D), lambda b: (b, 0)),
            pl.BlockSpec((1, 1, BLK), lambda b: (b, 0, 0)),
            pl.BlockSpec((K, D), lambda b: (0, 0)),
            pl.BlockSpec((NB, 1, BLK), lambda b: (0, 0, 0)),
            pl.BlockSpec(memory_space=pl.ANY),
        ],
        out_specs=pl.BlockSpec((1, 1), lambda b: (0, 0)),
        out_shape=jax.ShapeDtypeStruct((1, 1), jnp.float32),
        scratch_shapes=[
            pltpu.VMEM((NB, 1, BLK), jnp.float32),
            pltpu.VMEM((BPAD, D), jnp.float32),
            pltpu.SemaphoreType.DMA,
        ],
    )(node_fea, labs, center_fea, wts, F)
    return out.reshape(1)
